# baseline (speedup 1.0000x reference)
import functools

import jax
import jax.numpy as jnp
from jax.experimental import pallas as pl
from jax.experimental.pallas import tpu as pltpu

_EPS = 1e-6
_BN_EPS = 1e-5
_N_KNN = 50


def _pair_terms(xb, xtv, wrow):
    xm = [xb[:, c:c + 1] for c in range(3)]
    xn = [xtv[c:c + 1, :] for c in range(3)]
    c0 = [xm[c] - xn[c] for c in range(3)]
    cr = [
        xm[1] * xn[2] - xm[2] * xn[1],
        xm[2] * xn[0] - xm[0] * xn[2],
        xm[0] * xn[1] - xm[1] * xn[0],
    ]
    out = []
    for j in range(3):
        w0, w1, w2 = wrow[j]
        out.append([w0 * c0[c] + (w1 * xn[c] + w2 * cr[c]) for c in range(3)])
    return out


def _body(x_ref, xt_ref, wf_ref, wd_ref, out_ref, thr_ref, acc_ref, dint_ref,
          *, n, k, cnt_total, mb):
    ph = pl.program_id(0)
    b = pl.program_id(1)

    xtv = xt_ref[0]
    xx_n = jnp.sum(xtv * xtv, axis=0, keepdims=True)

    xt16 = xtv.astype(jnp.bfloat16)

    def _dbuild(i, carry):
        j0 = i * mb
        xb = x_ref[0, pl.ds(j0, mb), :]
        gram = jnp.dot(xb.astype(jnp.bfloat16), xt16,
                       preferred_element_type=jnp.float32)
        xx_m = jnp.sum(xb * xb, axis=1, keepdims=True)
        inner = -2.0 * gram
        pair = (-xx_n - inner) - xx_m
        dist = jnp.maximum(-pair, 0.0)
        dint_ref[pl.ds(j0, mb), :] = jax.lax.bitcast_convert_type(dist, jnp.int32)
        return carry

    jax.lax.fori_loop(0, n // mb, _dbuild, 0)

    wf = [[wf_ref[j, i] for i in range(3)] for j in range(3)]
    wd = [[wd_ref[j, i] for i in range(3)] for j in range(3)]

    @pl.when(jnp.logical_and(ph == 0, b == 0))
    def _init():
        for i in range(6):
            acc_ref[i] = 0.0

    @pl.when(ph == 0)
    def _phase0():
        lo0 = jnp.zeros((1, n), jnp.int32)
        hi0 = jnp.full((1, n), jnp.int32(0x7F800000))

        def step(_, carry):
            lo, hi = carry
            mid = lo + jax.lax.shift_right_logical(hi - lo, 1)
            cnt = jnp.sum((dint_ref[...] <= mid).astype(jnp.int32),
                          axis=0, keepdims=True)
            sel = cnt >= k
            return jnp.where(sel, lo, mid + 1), jnp.where(sel, mid, hi)

        lo, hi = jax.lax.fori_loop(0, 31, step, (lo0, hi0))
        thr = hi
        thr_ref[pl.ds(b, 1), 0:1, :] = thr.reshape(1, 1, n)

        def _sblock(i, carry):
            j0 = i * mb
            maskf = (dint_ref[pl.ds(j0, mb), :] <= thr).astype(jnp.float32)
            xb = x_ref[0, pl.ds(j0, mb), :]
            p = _pair_terms(xb, xtv, wf)
            a1, a2 = carry
            a1n, a2n = [], []
            for j in range(3):
                nrm = jnp.sqrt(p[j][0] * p[j][0] + p[j][1] * p[j][1]
                               + p[j][2] * p[j][2]) + _EPS
                mnrm = maskf * nrm
                a1n.append(a1[j] + jnp.sum(mnrm, axis=0, keepdims=True))
                a2n.append(a2[j] + jnp.sum(mnrm * nrm, axis=0, keepdims=True))
            return tuple(a1n), tuple(a2n)

        z3 = tuple(jnp.zeros((1, n), jnp.float32) for _ in range(3))
        acc1, acc2 = jax.lax.fori_loop(0, n // mb, _sblock, (z3, z3))
        for j in range(3):
            acc_ref[j] = acc_ref[j] + jnp.sum(acc1[j])
            acc_ref[3 + j] = acc_ref[3 + j] + jnp.sum(acc2[j])

    @pl.when(ph == 1)
    def _phase1():
        mean = [acc_ref[j] / cnt_total for j in range(3)]
        inv = [jax.lax.rsqrt(acc_ref[3 + j] / cnt_total - mean[j] * mean[j]
                             + _BN_EPS) for j in range(3)]
        thr = thr_ref[pl.ds(b, 1), 0:1, :].reshape(1, n)

        def _oblock(i, ov):
            j0 = i * mb
            maskf = (dint_ref[pl.ds(j0, mb), :] <= thr).astype(jnp.float32)
            xb = x_ref[0, pl.ds(j0, mb), :]
            p = _pair_terms(xb, xtv, wf)
            d = _pair_terms(xb, xtv, wd)
            ovn = []
            for j in range(3):
                nrm = jnp.sqrt(p[j][0] * p[j][0] + p[j][1] * p[j][1]
                               + p[j][2] * p[j][2]) + _EPS
                sfac = (nrm - mean[j]) * inv[j] / nrm
                pbn = [p[j][c] * sfac for c in range(3)]
                dot = pbn[0] * d[j][0] + pbn[1] * d[j][1] + pbn[2] * d[j][2]
                dsq = (d[j][0] * d[j][0] + d[j][1] * d[j][1]
                       + d[j][2] * d[j][2])
                coef = jnp.where(dot < 0.0, dot / (dsq + _EPS), 0.0)
                for c in range(3):
                    val = pbn[c] - coef * d[j][c]
                    ovn.append(ov[3 * j + c]
                               + jnp.sum(maskf * val, axis=0, keepdims=True))
            return tuple(ovn)

        ov = jax.lax.fori_loop(
            0, n // mb, _oblock,
            tuple(jnp.zeros((1, n), jnp.float32) for _ in range(9)))

        scale = 1.0 / (float(k) * float(n))
        row = jax.lax.broadcasted_iota(jnp.int32, (8, 128), 0)
        lane = jax.lax.broadcasted_iota(jnp.int32, (8, 128), 1)
        res = jnp.zeros((8, 128), jnp.float32)
        for j in range(3):
            for c in range(3):
                s = jnp.sum(ov[3 * j + c]) * scale
                res = jnp.where(jnp.logical_and(row == j, lane == c), s, res)
        out_ref[0, 0] = res

    @pl.when(ph == 0)
    def _zero_out():
        out_ref[0, 0] = jnp.zeros((8, 128), jnp.float32)


def kernel(x, W_feat, W_dir):
    B, N, _ = x.shape
    k = min(_N_KNN, N - 1)
    xt = jnp.transpose(x, (0, 2, 1))
    mb = 256 if N % 256 == 0 else N
    body = functools.partial(_body, n=N, k=k, cnt_total=float(B * N * k), mb=mb)
    out = pl.pallas_call(
        body,
        grid=(2, B),
        in_specs=[
            pl.BlockSpec((1, N, 3), lambda ph, b: (b, 0, 0)),
            pl.BlockSpec((1, 3, N), lambda ph, b: (b, 0, 0)),
            pl.BlockSpec(memory_space=pltpu.SMEM),
            pl.BlockSpec(memory_space=pltpu.SMEM),
        ],
        out_specs=pl.BlockSpec((1, 1, 8, 128), lambda ph, b: (ph, b, 0, 0)),
        out_shape=jax.ShapeDtypeStruct((2, B, 8, 128), jnp.float32),
        scratch_shapes=[
            pltpu.VMEM((B, 8, N), jnp.int32),
            pltpu.SMEM((8,), jnp.float32),
            pltpu.VMEM((N, N), jnp.int32),
        ],
    )(x, xt, W_feat, W_dir)
    return out[1, :, :3, :3]


# speedup vs baseline: 1.0321x; 1.0321x over previous
import functools

import jax
import jax.numpy as jnp
from jax.experimental import pallas as pl
from jax.experimental.pallas import tpu as pltpu

_EPS = 1e-6
_BN_EPS = 1e-5
_N_KNN = 50


def _pair_terms(xb, xtv, wrow):
    xm = [xb[:, c:c + 1] for c in range(3)]
    xn = [xtv[c:c + 1, :] for c in range(3)]
    c0 = [xm[c] - xn[c] for c in range(3)]
    cr = [
        xm[1] * xn[2] - xm[2] * xn[1],
        xm[2] * xn[0] - xm[0] * xn[2],
        xm[0] * xn[1] - xm[1] * xn[0],
    ]
    out = []
    for j in range(3):
        w0, w1, w2 = wrow[j]
        out.append([w0 * c0[c] + (w1 * xn[c] + w2 * cr[c]) for c in range(3)])
    return out


def _body(x_ref, xt_ref, wf_ref, wd_ref, out_ref, thr_ref, acc_ref, dint_ref,
          *, n, k, cnt_total, mb):
    ph = pl.program_id(0)
    b = pl.program_id(1)

    xtv = xt_ref[0]
    xx_n = jnp.sum(xtv * xtv, axis=0, keepdims=True)

    xt16 = xtv.astype(jnp.bfloat16)

    def _dbuild(i, carry):
        j0 = i * mb
        xb = x_ref[0, pl.ds(j0, mb), :]
        gram = jnp.dot(xb.astype(jnp.bfloat16), xt16,
                       preferred_element_type=jnp.float32)
        xx_m = jnp.sum(xb * xb, axis=1, keepdims=True)
        inner = -2.0 * gram
        pair = (-xx_n - inner) - xx_m
        dist = jnp.maximum(-pair, 0.0)
        dint_ref[pl.ds(j0, mb), :] = jax.lax.bitcast_convert_type(dist, jnp.int32)
        return carry

    jax.lax.fori_loop(0, n // mb, _dbuild, 0)

    wf = [[wf_ref[j, i] for i in range(3)] for j in range(3)]
    wd = [[wd_ref[j, i] for i in range(3)] for j in range(3)]

    @pl.when(jnp.logical_and(ph == 0, b == 0))
    def _init():
        for i in range(6):
            acc_ref[i] = 0.0

    @pl.when(ph == 0)
    def _phase0():
        lo0 = jnp.zeros((1, n), jnp.int32)
        hi0 = jnp.full((1, n), jnp.int32(0x7F800000))
        act0 = jnp.ones((1, n), jnp.int32)

        def cond(carry):
            i, _, _, _, act = carry
            return jnp.logical_and(i < 31, jnp.sum(act) > 0)

        def step(carry):
            i, lo, hi, thr, act = carry
            mid = lo + jax.lax.shift_right_logical(hi - lo, 1)
            cnt = jnp.sum((dint_ref[...] <= mid).astype(jnp.int32),
                          axis=0, keepdims=True)
            eq = (cnt == k).astype(jnp.int32)
            hit = act * eq
            thr = jnp.where(hit > 0, mid, thr)
            act = act * (1 - eq)
            sel = cnt >= k
            return (i + 1, jnp.where(sel, lo, mid + 1),
                    jnp.where(sel, mid, hi), thr, act)

        _, lo, hi, thr, act = jax.lax.while_loop(
            cond, step, (0, lo0, hi0, jnp.zeros((1, n), jnp.int32), act0))
        thr = jnp.where(act > 0, hi, thr)
        thr_ref[pl.ds(b, 1), 0:1, :] = thr.reshape(1, 1, n)

        def _sblock(i, carry):
            j0 = i * mb
            maskf = (dint_ref[pl.ds(j0, mb), :] <= thr).astype(jnp.float32)
            xb = x_ref[0, pl.ds(j0, mb), :]
            p = _pair_terms(xb, xtv, wf)
            a1, a2 = carry
            a1n, a2n = [], []
            for j in range(3):
                nrm = jnp.sqrt(p[j][0] * p[j][0] + p[j][1] * p[j][1]
                               + p[j][2] * p[j][2]) + _EPS
                mnrm = maskf * nrm
                a1n.append(a1[j] + jnp.sum(mnrm, axis=0, keepdims=True))
                a2n.append(a2[j] + jnp.sum(mnrm * nrm, axis=0, keepdims=True))
            return tuple(a1n), tuple(a2n)

        z3 = tuple(jnp.zeros((1, n), jnp.float32) for _ in range(3))
        acc1, acc2 = jax.lax.fori_loop(0, n // mb, _sblock, (z3, z3))
        for j in range(3):
            acc_ref[j] = acc_ref[j] + jnp.sum(acc1[j])
            acc_ref[3 + j] = acc_ref[3 + j] + jnp.sum(acc2[j])

    @pl.when(ph == 1)
    def _phase1():
        mean = [acc_ref[j] / cnt_total for j in range(3)]
        inv = [jax.lax.rsqrt(acc_ref[3 + j] / cnt_total - mean[j] * mean[j]
                             + _BN_EPS) for j in range(3)]
        thr = thr_ref[pl.ds(b, 1), 0:1, :].reshape(1, n)

        def _oblock(i, ov):
            j0 = i * mb
            maskf = (dint_ref[pl.ds(j0, mb), :] <= thr).astype(jnp.float32)
            xb = x_ref[0, pl.ds(j0, mb), :]
            p = _pair_terms(xb, xtv, wf)
            d = _pair_terms(xb, xtv, wd)
            ovn = []
            for j in range(3):
                nrm = jnp.sqrt(p[j][0] * p[j][0] + p[j][1] * p[j][1]
                               + p[j][2] * p[j][2]) + _EPS
                sfac = (nrm - mean[j]) * inv[j] / nrm
                pbn = [p[j][c] * sfac for c in range(3)]
                dot = pbn[0] * d[j][0] + pbn[1] * d[j][1] + pbn[2] * d[j][2]
                dsq = (d[j][0] * d[j][0] + d[j][1] * d[j][1]
                       + d[j][2] * d[j][2])
                coef = jnp.where(dot < 0.0, dot / (dsq + _EPS), 0.0)
                for c in range(3):
                    val = pbn[c] - coef * d[j][c]
                    ovn.append(ov[3 * j + c]
                               + jnp.sum(maskf * val, axis=0, keepdims=True))
            return tuple(ovn)

        ov = jax.lax.fori_loop(
            0, n // mb, _oblock,
            tuple(jnp.zeros((1, n), jnp.float32) for _ in range(9)))

        scale = 1.0 / (float(k) * float(n))
        row = jax.lax.broadcasted_iota(jnp.int32, (8, 128), 0)
        lane = jax.lax.broadcasted_iota(jnp.int32, (8, 128), 1)
        res = jnp.zeros((8, 128), jnp.float32)
        for j in range(3):
            for c in range(3):
                s = jnp.sum(ov[3 * j + c]) * scale
                res = jnp.where(jnp.logical_and(row == j, lane == c), s, res)
        out_ref[0, 0] = res

    @pl.when(ph == 0)
    def _zero_out():
        out_ref[0, 0] = jnp.zeros((8, 128), jnp.float32)


def kernel(x, W_feat, W_dir):
    B, N, _ = x.shape
    k = min(_N_KNN, N - 1)
    xt = jnp.transpose(x, (0, 2, 1))
    mb = 128 if N % 128 == 0 else N
    body = functools.partial(_body, n=N, k=k, cnt_total=float(B * N * k), mb=mb)
    out = pl.pallas_call(
        body,
        grid=(2, B),
        in_specs=[
            pl.BlockSpec((1, N, 3), lambda ph, b: (b, 0, 0)),
            pl.BlockSpec((1, 3, N), lambda ph, b: (b, 0, 0)),
            pl.BlockSpec(memory_space=pltpu.SMEM),
            pl.BlockSpec(memory_space=pltpu.SMEM),
        ],
        out_specs=pl.BlockSpec((1, 1, 8, 128), lambda ph, b: (ph, b, 0, 0)),
        out_shape=jax.ShapeDtypeStruct((2, B, 8, 128), jnp.float32),
        scratch_shapes=[
            pltpu.VMEM((B, 8, N), jnp.int32),
            pltpu.SMEM((8,), jnp.float32),
            pltpu.VMEM((N, N), jnp.int32),
        ],
    )(x, xt, W_feat, W_dir)
    return out[1, :, :3, :3]
